# Initial kernel scaffold; baseline (speedup 1.0000x reference)
import jax
import jax.numpy as jnp
from jax.experimental import pallas as pl


def kernel(pred_logits, pred_boxes, target_sizes):
    raise NotImplementedError("write your pallas kernel here")


# speedup vs baseline: 7.3433x; 7.3433x over previous
import functools

import jax
import jax.numpy as jnp
from jax import lax
from jax.experimental import pallas as pl
from jax.experimental.pallas import tpu as pltpu
from jax.experimental.pallas import tpu_sc as plsc

B = 64
Q = 900
C = 91
N = Q * C
NPAD = 81920
K = 100
KPAD = 112
CHUNK = 256
NCHUNK = NPAD // CHUNK
L = 16

_NEG_INF = float("-inf")
_BIG = 1 << 30


def _tile_body(logits_hbm, boxes_hbm, ts_hbm,
               scores_hbm, labels_hbm, boxes_out_hbm,
               x_v, cm_v, vals_v, idx_v,
               scores_v, labels_v, brow_v, bout_v, ts_v):
    wid = lax.axis_index("s") * 2 + lax.axis_index("c")
    lanes = lax.iota(jnp.int32, L)
    lane0 = lanes == 0

    pltpu.sync_copy(ts_hbm, ts_v)

    for r2 in range(2):
        row = wid * 2 + r2

        pltpu.sync_copy(logits_hbm.at[row], x_v)
        pltpu.sync_copy(boxes_hbm.at[row], brow_v)

        def chunk_max(c, _):
            m = jnp.full((L,), _NEG_INF, jnp.float32)
            base = c * CHUNK
            for j in range(CHUNK // L):
                m = jnp.maximum(m, x_v[pl.ds(base + j * L, L)])
            cmax = jnp.max(m)
            plsc.store_scatter(cm_v, [jnp.full((L,), c, jnp.int32)],
                               jnp.full((L,), cmax, jnp.float32),
                               mask=lane0)
            return 0

        lax.fori_loop(0, NCHUNK, chunk_max, 0)

        def extract(e, _):
            m = cm_v[pl.ds(0, L)]
            for g in range(1, NCHUNK // L):
                m = jnp.maximum(m, cm_v[pl.ds(g * L, L)])
            gmax = jnp.max(m)
            best = jnp.full((L,), _BIG, jnp.int32)
            for g in range(NCHUNK // L):
                eq = cm_v[pl.ds(g * L, L)] == gmax
                best = jnp.minimum(best, jnp.where(eq, g * L + lanes, _BIG))
            c_star = jnp.min(best)
            cbase = c_star * CHUNK
            best2 = jnp.full((L,), _BIG, jnp.int32)
            for j in range(CHUNK // L):
                eq = x_v[pl.ds(cbase + j * L, L)] == gmax
                best2 = jnp.minimum(best2, jnp.where(eq, j * L + lanes, _BIG))
            pos = jnp.min(best2)
            flat = cbase + pos

            e_splat = jnp.full((L,), e, jnp.int32)
            plsc.store_scatter(vals_v, [e_splat],
                               jnp.full((L,), gmax, jnp.float32), mask=lane0)
            plsc.store_scatter(idx_v, [e_splat],
                               jnp.full((L,), flat, jnp.int32), mask=lane0)

            vbase = cbase + (pos // L) * L
            v = x_v[pl.ds(vbase, L)]
            x_v[pl.ds(vbase, L)] = jnp.where(lanes == pos % L, _NEG_INF, v)
            m2 = jnp.full((L,), _NEG_INF, jnp.float32)
            for j in range(CHUNK // L):
                m2 = jnp.maximum(m2, x_v[pl.ds(cbase + j * L, L)])
            plsc.store_scatter(cm_v, [jnp.full((L,), c_star, jnp.int32)],
                               jnp.full((L,), jnp.max(m2), jnp.float32),
                               mask=lane0)
            return 0

        lax.fori_loop(0, K, extract, 0)

        tbase = (row // 8) * L
        tsv = ts_v[pl.ds(tbase, L)]
        toff = row * 2 - tbase
        hf = jnp.max(jnp.where(lanes == toff, tsv, -1)).astype(jnp.float32)
        wf = jnp.max(jnp.where(lanes == toff + 1, tsv, -1)).astype(jnp.float32)
        zero = jnp.zeros((L,), jnp.int32)
        for g in range(KPAD // L):
            v = vals_v[pl.ds(g * L, L)]
            scores_v[pl.ds(g * L, L)] = 1.0 / (1.0 + jnp.exp(-v))
            fi = idx_v[pl.ds(g * L, L)]
            labels_v[pl.ds(g * L, L)] = fi % C
            q4 = jnp.clip(fi // C, 0, Q - 1) * 4
            cx = plsc.load_gather(brow_v, [q4])
            cy = plsc.load_gather(brow_v, [q4 + 1])
            w = plsc.load_gather(brow_v, [q4 + 2])
            h = plsc.load_gather(brow_v, [q4 + 3])
            ei4 = (g * L + lanes) * 4
            plsc.store_scatter(bout_v, [ei4], (cx - 0.5 * w) * wf)
            plsc.store_scatter(bout_v, [ei4 + 1], (cy - 0.5 * h) * hf)
            plsc.store_scatter(bout_v, [ei4 + 2], (cx + 0.5 * w) * wf)
            plsc.store_scatter(bout_v, [ei4 + 3], (cy + 0.5 * h) * hf)

        pltpu.sync_copy(scores_v, scores_hbm.at[row])
        pltpu.sync_copy(labels_v, labels_hbm.at[row])
        pltpu.sync_copy(bout_v, boxes_out_hbm.at[row])


_mesh = plsc.VectorSubcoreMesh(core_axis_name="c", subcore_axis_name="s")

_sc_call = functools.partial(
    pl.kernel,
    out_type=[
        jax.ShapeDtypeStruct((B, KPAD), jnp.float32),
        jax.ShapeDtypeStruct((B, KPAD), jnp.int32),
        jax.ShapeDtypeStruct((B, KPAD * 4), jnp.float32),
    ],
    mesh=_mesh,
    compiler_params=pltpu.CompilerParams(needs_layout_passes=False),
    scratch_types=[
        pltpu.VMEM((NPAD,), jnp.float32),
        pltpu.VMEM((NCHUNK,), jnp.float32),
        pltpu.VMEM((KPAD,), jnp.float32),
        pltpu.VMEM((KPAD,), jnp.int32),
        pltpu.VMEM((KPAD,), jnp.float32),
        pltpu.VMEM((KPAD,), jnp.int32),
        pltpu.VMEM((Q * 4,), jnp.float32),
        pltpu.VMEM((KPAD * 4,), jnp.float32),
        pltpu.VMEM((B * 2,), jnp.int32),
    ],
)(_tile_body)


@jax.jit
def kernel(pred_logits, pred_boxes, target_sizes):
    lp = pred_logits.reshape(B, N)
    lp = jnp.pad(lp, ((0, 0), (0, NPAD - N)), constant_values=-jnp.inf)
    scores_p, labels_p, boxes_p = _sc_call(lp, pred_boxes.reshape(B, Q * 4),
                                           target_sizes.reshape(B * 2))
    return (scores_p[:, :K], labels_p[:, :K],
            boxes_p.reshape(B, KPAD, 4)[:, :K, :])
